# baseline (speedup 1.0000x reference)
import functools

import jax
import jax.numpy as jnp
from jax import lax
from jax.experimental import pallas as pl
from jax.experimental.pallas import tpu as pltpu
from jax.experimental.pallas import tpu_sc as plsc

_B = 16384
_DIM = 32
_NC = 2
_NS = 16
_NW = _NC * _NS
_BPW = _B // _NW
_CHUNK = 128
_NCHUNK = _BPW // _CHUNK
_L = 16


def _sc_body(src_hbm, tgt_hbm, ns_hbm, ctx_hbm, out_hbm,
             sidx, tidx, srows, trows, outv, sem):
    wid = lax.axis_index("s") * _NC + lax.axis_index("c")
    base = wid * _BPW

    for k in range(_NCHUNK):
        pltpu.sync_copy(src_hbm.at[pl.ds(base + k * _CHUNK, _CHUNK)], sidx.at[k])
        pltpu.sync_copy(tgt_hbm.at[pl.ds(base + k * _CHUNK, _CHUNK)], tidx.at[k])

    copies = []
    for k in range(_NCHUNK):
        copies.append(pltpu.async_copy(
            ns_hbm.at[sidx.at[k]], srows.at[pl.ds(k * _CHUNK, _CHUNK)], sem))
        copies.append(pltpu.async_copy(
            ctx_hbm.at[tidx.at[k]], trows.at[pl.ds(k * _CHUNK, _CHUNK)], sem))
    for c in copies:
        c.wait()

    lane = lax.iota(jnp.int32, _L)

    def group(g, carry):
        rows = g * _L + lane
        acc = jnp.zeros((_L,), jnp.float32)
        for d in range(_DIM):
            col = jnp.full((_L,), d, jnp.int32)
            sv = plsc.load_gather(srows, [rows, col])
            tv = plsc.load_gather(trows, [rows, col])
            acc = acc + sv * tv
        outv[pl.ds(pl.multiple_of(g * _L, _L), _L)] = acc
        return carry

    lax.fori_loop(0, _BPW // _L, group, 0)

    pltpu.sync_copy(outv, out_hbm.at[pl.ds(base, _BPW)])


@functools.partial(
    pl.kernel,
    out_type=jax.ShapeDtypeStruct((_B,), jnp.float32),
    mesh=plsc.VectorSubcoreMesh(core_axis_name="c", subcore_axis_name="s"),
    scratch_types=[
        pltpu.VMEM((_NCHUNK, _CHUNK), jnp.int32),
        pltpu.VMEM((_NCHUNK, _CHUNK), jnp.int32),
        pltpu.VMEM((_BPW, _DIM), jnp.float32),
        pltpu.VMEM((_BPW, _DIM), jnp.float32),
        pltpu.VMEM((_BPW,), jnp.float32),
        pltpu.SemaphoreType.DMA,
    ],
    compiler_params=pltpu.CompilerParams(
        needs_layout_passes=False, use_tc_tiling_on_sc=False),
)
def _sc_dot(src_hbm, tgt_hbm, ns_hbm, ctx_hbm, out_hbm,
            sidx, tidx, srows, trows, outv, sem):
    _sc_body(src_hbm, tgt_hbm, ns_hbm, ctx_hbm, out_hbm,
             sidx, tidx, srows, trows, outv, sem)


def _loss_body(ip_ref, lab_ref, o_ref):
    x = lab_ref[...] * ip_ref[...]
    o_ref[0, 0] = -jnp.sum(jax.nn.log_sigmoid(x)) * (1.0 / _B)


_loss = pl.pallas_call(
    _loss_body,
    out_shape=jax.ShapeDtypeStruct((1, 1), jnp.float32),
    out_specs=pl.BlockSpec(memory_space=pltpu.MemorySpace.SMEM),
)


def kernel(source_node, target_node, label, nodes_embed, context_nodes_embed):
    ip = _sc_dot(source_node, target_node, nodes_embed, context_nodes_embed)
    loss = _loss(ip.reshape(128, 128), label.reshape(128, 128))
    return loss.reshape(())
